# baseline (speedup 1.0000x reference)
import functools

import jax
import jax.numpy as jnp
from jax import lax
from jax.experimental import pallas as pl
from jax.experimental.pallas import tpu as pltpu
from jax.experimental.pallas import tpu_sc as plsc

_VOCAB = 100000
_D = 768
_BATCH = 4
_SEQ = 2048
_NC = 2
_NS = 16
_NW = _NC * _NS
_POS_W = _SEQ // _NW
_PSUB = 8
_NSUB = _POS_W // _PSUB
_ROWS = _BATCH * _PSUB
_NBUF = 4
_LANES = 16
_VECS = _D // _LANES


def _emb_body(ids_hbm, tok_hbm, pos_hbm, out_hbm,
              ids_stage, idx_v,
              tv0, tv1, tv2, tv3, pv0, pv1, pv2, pv3,
              si0, si1, si2, si3, so0, so1, so2, so3):
    wid = lax.axis_index("c") * _NS + lax.axis_index("s")
    pos0 = wid * _POS_W

    tv = (tv0, tv1, tv2, tv3)
    pv = (pv0, pv1, pv2, pv3)
    s_in = (si0, si1, si2, si3)
    s_out = (so0, so1, so2, so3)

    stage_h = [
        pltpu.async_copy(ids_hbm.at[bat, pl.ds(pos0, _POS_W)],
                         ids_stage.at[pl.ds(bat * _POS_W, _POS_W)], si0)
        for bat in range(_BATCH)
    ]
    for hnd in stage_h:
        hnd.wait()
    for h in range(_NSUB):
        for bat in range(_BATCH):
            idx_v[pl.ds((h * _BATCH + bat) * _PSUB, _PSUB)] = (
                ids_stage[pl.ds(bat * _POS_W + h * _PSUB, _PSUB)])

    def fire(h, b):
        g = pltpu.async_copy(tok_hbm.at[idx_v.at[pl.ds(h * _ROWS, _ROWS)]],
                             tv[b], s_in[b])
        p = pltpu.async_copy(pos_hbm.at[pl.ds(pos0 + h * _PSUB, _PSUB)],
                             pv[b], s_in[b])
        return (g, p)

    in_h = [None] * _NBUF
    out_h = [[] for _ in range(_NBUF)]
    for h in range(_NBUF - 1):
        in_h[h] = fire(h, h)

    for h in range(_NSUB):
        b = h % _NBUF
        in_h[b][0].wait()
        in_h[b][1].wait()

        tvb, pvb = tv[b], pv[b]

        @plsc.parallel_loop(0, _PSUB * _VECS, 1, unroll=8)
        def _add(k):
            p = k & (_PSUB - 1)
            j = k >> 3
            s = pl.ds(j * _LANES, _LANES)
            pvec = pvb[p, s]
            for bat in range(_BATCH):
                r = bat * _PSUB + p
                tvb[r, s] = tvb[r, s] + pvec

        for bat in range(_BATCH):
            out_h[b].append(pltpu.async_copy(
                tvb.at[pl.ds(bat * _PSUB, _PSUB)],
                out_hbm.at[bat, pl.ds(pos0 + h * _PSUB, _PSUB)],
                s_out[b]))

        nxt = h + _NBUF - 1
        if nxt < _NSUB:
            nb = nxt % _NBUF
            for hnd in out_h[nb]:
                hnd.wait()
            out_h[nb] = []
            in_h[nb] = fire(nxt, nb)

    for hnds in out_h:
        for hnd in hnds:
            hnd.wait()


@jax.jit
def _emb(input_ids, token_table, pos_table):
    mesh = plsc.VectorSubcoreMesh(core_axis_name="c", subcore_axis_name="s")
    run = functools.partial(
        pl.kernel,
        mesh=mesh,
        out_type=jax.ShapeDtypeStruct((_BATCH, _SEQ, _D), jnp.float32),
        scratch_types=[
            pltpu.VMEM((_BATCH * _POS_W,), jnp.int32),
            pltpu.VMEM((_BATCH * _POS_W,), jnp.int32),
            pltpu.VMEM((_ROWS, _D), jnp.float32),
            pltpu.VMEM((_ROWS, _D), jnp.float32),
            pltpu.VMEM((_ROWS, _D), jnp.float32),
            pltpu.VMEM((_ROWS, _D), jnp.float32),
            pltpu.VMEM((_PSUB, _D), jnp.float32),
            pltpu.VMEM((_PSUB, _D), jnp.float32),
            pltpu.VMEM((_PSUB, _D), jnp.float32),
            pltpu.VMEM((_PSUB, _D), jnp.float32),
            pltpu.SemaphoreType.DMA,
            pltpu.SemaphoreType.DMA,
            pltpu.SemaphoreType.DMA,
            pltpu.SemaphoreType.DMA,
            pltpu.SemaphoreType.DMA,
            pltpu.SemaphoreType.DMA,
            pltpu.SemaphoreType.DMA,
            pltpu.SemaphoreType.DMA,
        ],
    )(_emb_body)
    return run(input_ids, token_table, pos_table)


def kernel(input_ids, token_table, pos_table):
    return _emb(input_ids.astype(jnp.int32), token_table, pos_table)


# speedup vs baseline: 1.0268x; 1.0268x over previous
import functools

import jax
import jax.numpy as jnp
from jax import lax
from jax.experimental import pallas as pl
from jax.experimental.pallas import tpu as pltpu
from jax.experimental.pallas import tpu_sc as plsc

_VOCAB = 100000
_D = 768
_BATCH = 4
_SEQ = 2048
_NC = 2
_NS = 16
_NW = _NC * _NS
_POS_W = _SEQ // _NW
_PSUB = 8
_NSUB = _POS_W // _PSUB
_ROWS = _BATCH * _PSUB
_NBUF = 4
_LANES = 16
_VECS = _D // _LANES


def _emb_body(ids_hbm, tok_hbm, pos_hbm, out_hbm,
              ids_stage, idx_v,
              tv0, tv1, tv2, tv3, pv0, pv1, pv2, pv3,
              si0, si1, si2, si3, so0, so1, so2, so3):
    wid = lax.axis_index("c") * _NS + lax.axis_index("s")
    pos0 = wid * _POS_W

    tv = (tv0, tv1, tv2, tv3)
    pv = (pv0, pv1, pv2, pv3)
    s_in = (si0, si1, si2, si3)
    s_out = (so0, so1, so2, so3)

    stage_h = [
        pltpu.async_copy(ids_hbm.at[bat, pl.ds(pos0, _POS_W)],
                         ids_stage.at[pl.ds(bat * _POS_W, _POS_W)], si0)
        for bat in range(_BATCH)
    ]
    for hnd in stage_h:
        hnd.wait()
    for h in range(_NSUB):
        for bat in range(_BATCH):
            idx_v[pl.ds((h * _BATCH + bat) * _PSUB, _PSUB)] = (
                ids_stage[pl.ds(bat * _POS_W + h * _PSUB, _PSUB)])

    def fire(h, b):
        g = pltpu.async_copy(tok_hbm.at[idx_v.at[pl.ds(h * _ROWS, _ROWS)]],
                             tv[b], s_in[b])
        p = pltpu.async_copy(pos_hbm.at[pl.ds(pos0 + h * _PSUB, _PSUB)],
                             pv[b], s_in[b])
        return (g, p)

    in_h = [None] * _NBUF
    out_h = [[] for _ in range(_NBUF)]
    for h in range(_NBUF - 1):
        in_h[h] = fire(h, h)

    for h in range(_NSUB):
        b = h % _NBUF
        nxt = h + _NBUF - 1
        if nxt < _NSUB:
            nb = nxt % _NBUF
            for hnd in out_h[nb]:
                hnd.wait()
            out_h[nb] = []
            in_h[nb] = fire(nxt, nb)
        in_h[b][0].wait()
        in_h[b][1].wait()

        tvb, pvb = tv[b], pv[b]

        @plsc.parallel_loop(0, _PSUB * _VECS, 1, unroll=4)
        def _add(k):
            p = k & (_PSUB - 1)
            j = k >> 3
            s = pl.ds(j * _LANES, _LANES)
            pvec = pvb[p, s]
            for bat in range(_BATCH):
                r = bat * _PSUB + p
                tvb[r, s] = tvb[r, s] + pvec

        for bat in range(_BATCH):
            out_h[b].append(pltpu.async_copy(
                tvb.at[pl.ds(bat * _PSUB, _PSUB)],
                out_hbm.at[bat, pl.ds(pos0 + h * _PSUB, _PSUB)],
                s_out[b]))

    for hnds in out_h:
        for hnd in hnds:
            hnd.wait()


@jax.jit
def _emb(input_ids, token_table, pos_table):
    mesh = plsc.VectorSubcoreMesh(core_axis_name="c", subcore_axis_name="s")
    run = functools.partial(
        pl.kernel,
        mesh=mesh,
        out_type=jax.ShapeDtypeStruct((_BATCH, _SEQ, _D), jnp.float32),
        scratch_types=[
            pltpu.VMEM((_BATCH * _POS_W,), jnp.int32),
            pltpu.VMEM((_BATCH * _POS_W,), jnp.int32),
            pltpu.VMEM((_ROWS, _D), jnp.float32),
            pltpu.VMEM((_ROWS, _D), jnp.float32),
            pltpu.VMEM((_ROWS, _D), jnp.float32),
            pltpu.VMEM((_ROWS, _D), jnp.float32),
            pltpu.VMEM((_PSUB, _D), jnp.float32),
            pltpu.VMEM((_PSUB, _D), jnp.float32),
            pltpu.VMEM((_PSUB, _D), jnp.float32),
            pltpu.VMEM((_PSUB, _D), jnp.float32),
            pltpu.SemaphoreType.DMA,
            pltpu.SemaphoreType.DMA,
            pltpu.SemaphoreType.DMA,
            pltpu.SemaphoreType.DMA,
            pltpu.SemaphoreType.DMA,
            pltpu.SemaphoreType.DMA,
            pltpu.SemaphoreType.DMA,
            pltpu.SemaphoreType.DMA,
        ],
    )(_emb_body)
    return run(input_ids, token_table, pos_table)


def kernel(input_ids, token_table, pos_table):
    return _emb(input_ids.astype(jnp.int32), token_table, pos_table)


# speedup vs baseline: 1.0338x; 1.0069x over previous
import functools

import jax
import jax.numpy as jnp
from jax import lax
from jax.experimental import pallas as pl
from jax.experimental.pallas import tpu as pltpu
from jax.experimental.pallas import tpu_sc as plsc

_VOCAB = 100000
_D = 768
_BATCH = 4
_SEQ = 2048
_NC = 2
_NS = 16
_NW = _NC * _NS
_POS_W = _SEQ // _NW
_PSUB = 8
_NSUB = _POS_W // _PSUB
_ROWS = _BATCH * _PSUB
_NBUF = 4
_LANES = 16
_VECS = _D // _LANES


def _emb_body(ids_hbm, tok_hbm, pos_hbm, out_hbm,
              ids_stage, idx_v,
              tv0, tv1, tv2, tv3, pv0, pv1, pv2, pv3,
              si0, si1, si2, si3, so0, so1, so2, so3):
    wid = lax.axis_index("c") * _NS + lax.axis_index("s")
    pos0 = wid * _POS_W

    tv = (tv0, tv1, tv2, tv3)
    pv = (pv0, pv1, pv2, pv3)
    s_in = (si0, si1, si2, si3)
    s_out = (so0, so1, so2, so3)

    stage_h = [
        pltpu.async_copy(ids_hbm.at[bat, pl.ds(pos0, _POS_W)],
                         ids_stage.at[pl.ds(bat * _POS_W, _POS_W)], si0)
        for bat in range(_BATCH)
    ]
    for hnd in stage_h:
        hnd.wait()
    for h in range(_NSUB):
        for bat in range(_BATCH):
            idx_v[pl.ds((h * _BATCH + bat) * _PSUB, _PSUB)] = (
                ids_stage[pl.ds(bat * _POS_W + h * _PSUB, _PSUB)])

    def fire(h, b):
        g = pltpu.async_copy(tok_hbm.at[idx_v.at[pl.ds(h * _ROWS, _ROWS)]],
                             tv[b], s_in[b])
        p = pltpu.async_copy(pos_hbm.at[pl.ds(pos0 + h * _PSUB, _PSUB)],
                             pv[b], s_in[b])
        return (g, p)

    in_h = [None] * _NBUF
    out_h = [[] for _ in range(_NBUF)]
    for h in range(_NBUF - 1):
        in_h[h] = fire(h, h)

    for h in range(_NSUB):
        b = h % _NBUF
        nxt = h + _NBUF - 1
        if nxt < _NSUB:
            nb = nxt % _NBUF
            for hnd in out_h[nb]:
                hnd.wait()
            out_h[nb] = []
            in_h[nb] = fire(nxt, nb)
        in_h[b][0].wait()
        in_h[b][1].wait()

        tvb, pvb = tv[b], pv[b]

        @plsc.parallel_loop(0, _PSUB * _VECS, 1, unroll=2)
        def _add(k):
            p = k & (_PSUB - 1)
            j = k >> 3
            s = pl.ds(j * _LANES, _LANES)
            pvec = pvb[p, s]
            for bat in range(_BATCH):
                r = bat * _PSUB + p
                tvb[r, s] = tvb[r, s] + pvec

        for bat in range(_BATCH):
            out_h[b].append(pltpu.async_copy(
                tvb.at[pl.ds(bat * _PSUB, _PSUB)],
                out_hbm.at[bat, pl.ds(pos0 + h * _PSUB, _PSUB)],
                s_out[b]))

    for hnds in out_h:
        for hnd in hnds:
            hnd.wait()


@jax.jit
def _emb(input_ids, token_table, pos_table):
    mesh = plsc.VectorSubcoreMesh(core_axis_name="c", subcore_axis_name="s")
    run = functools.partial(
        pl.kernel,
        mesh=mesh,
        out_type=jax.ShapeDtypeStruct((_BATCH, _SEQ, _D), jnp.float32),
        scratch_types=[
            pltpu.VMEM((_BATCH * _POS_W,), jnp.int32),
            pltpu.VMEM((_BATCH * _POS_W,), jnp.int32),
            pltpu.VMEM((_ROWS, _D), jnp.float32),
            pltpu.VMEM((_ROWS, _D), jnp.float32),
            pltpu.VMEM((_ROWS, _D), jnp.float32),
            pltpu.VMEM((_ROWS, _D), jnp.float32),
            pltpu.VMEM((_PSUB, _D), jnp.float32),
            pltpu.VMEM((_PSUB, _D), jnp.float32),
            pltpu.VMEM((_PSUB, _D), jnp.float32),
            pltpu.VMEM((_PSUB, _D), jnp.float32),
            pltpu.SemaphoreType.DMA,
            pltpu.SemaphoreType.DMA,
            pltpu.SemaphoreType.DMA,
            pltpu.SemaphoreType.DMA,
            pltpu.SemaphoreType.DMA,
            pltpu.SemaphoreType.DMA,
            pltpu.SemaphoreType.DMA,
            pltpu.SemaphoreType.DMA,
        ],
    )(_emb_body)
    return run(input_ids, token_table, pos_table)


def kernel(input_ids, token_table, pos_table):
    return _emb(input_ids.astype(jnp.int32), token_table, pos_table)
